# Initial kernel scaffold; baseline (speedup 1.0000x reference)
import jax
import jax.numpy as jnp
from jax.experimental import pallas as pl


def kernel(x, edge_index, W1, b1, W2, b2):
    raise NotImplementedError("write your pallas kernel here")


# speedup vs baseline: 1.0423x; 1.0423x over previous
import functools

import jax
import jax.numpy as jnp
from jax.experimental import pallas as pl
from jax.experimental.pallas import tpu as pltpu

_N = 10000
_E = 320000
_D_IN = 128
_D_HID = 128
_D_OUT = 40
_D_PAD = 48

_ROWS = 1000


def _mm1_body(x_ref, ns_ref, w_ref, o_ref):
    ns = ns_ref[...]
    o_ref[...] = jnp.dot(x_ref[...] * ns, w_ref[...],
                         preferred_element_type=jnp.float32)


def _mm1(x, norm_src, W1):
    return pl.pallas_call(
        _mm1_body,
        grid=(_N // _ROWS,),
        in_specs=[
            pl.BlockSpec((_ROWS, _D_IN), lambda i: (i, 0)),
            pl.BlockSpec((_ROWS, 1), lambda i: (i, 0)),
            pl.BlockSpec((_D_IN, _D_HID), lambda i: (0, 0)),
        ],
        out_specs=pl.BlockSpec((_ROWS, _D_HID), lambda i: (i, 0)),
        out_shape=jax.ShapeDtypeStruct((_N, _D_HID), jnp.float32),
    )(x, norm_src, W1)


def _mm2_body(a_ref, nd_ref, ns_ref, b1_ref, w_ref, o_ref):
    h = jax.nn.relu(a_ref[...] * nd_ref[...] + b1_ref[...])
    o_ref[...] = jnp.dot(h * ns_ref[...], w_ref[...],
                         preferred_element_type=jnp.float32)


def _mm2(agg1, norm_dst, norm_src, b1, W2p):
    return pl.pallas_call(
        _mm2_body,
        grid=(_N // _ROWS,),
        in_specs=[
            pl.BlockSpec((_ROWS, _D_HID), lambda i: (i, 0)),
            pl.BlockSpec((_ROWS, 1), lambda i: (i, 0)),
            pl.BlockSpec((_ROWS, 1), lambda i: (i, 0)),
            pl.BlockSpec((1, _D_HID), lambda i: (0, 0)),
            pl.BlockSpec((_D_HID, _D_PAD), lambda i: (0, 0)),
        ],
        out_specs=pl.BlockSpec((_ROWS, _D_PAD), lambda i: (i, 0)),
        out_shape=jax.ShapeDtypeStruct((_N, _D_PAD), jnp.float32),
    )(agg1, norm_dst, norm_src, b1, W2p)


def _fin_body(a_ref, nd_ref, b2_ref, o_ref):
    o_ref[...] = a_ref[...] * nd_ref[...] + b2_ref[...]


def _fin(agg2, norm_dst, b2p):
    return pl.pallas_call(
        _fin_body,
        grid=(_N // _ROWS,),
        in_specs=[
            pl.BlockSpec((_ROWS, _D_PAD), lambda i: (i, 0)),
            pl.BlockSpec((_ROWS, 1), lambda i: (i, 0)),
            pl.BlockSpec((1, _D_PAD), lambda i: (0, 0)),
        ],
        out_specs=pl.BlockSpec((_ROWS, _D_PAD), lambda i: (i, 0)),
        out_shape=jax.ShapeDtypeStruct((_N, _D_PAD), jnp.float32),
    )(agg2, norm_dst, b2p)


def kernel(x, edge_index, W1, b1, W2, b2):
    src = edge_index[0]
    dst = edge_index[1]
    ones = jnp.ones((_E,), dtype=jnp.float32)
    deg_out = jax.ops.segment_sum(ones, src, num_segments=_N)
    deg_in = jax.ops.segment_sum(ones, dst, num_segments=_N)
    norm_src = jnp.where(deg_out > 0, deg_out**-0.5, 0.0)[:, None]
    norm_dst = jnp.where(deg_in > 0, deg_in**-0.5, 0.0)[:, None]

    h_pre = _mm1(x, norm_src, W1)
    agg1 = jax.ops.segment_sum(jnp.take(h_pre, src, axis=0), dst,
                               num_segments=_N)

    W2p = jnp.pad(W2, ((0, 0), (0, _D_PAD - _D_OUT)))
    y = _mm2(agg1, norm_dst, norm_src, b1[None, :], W2p)
    agg2 = jax.ops.segment_sum(jnp.take(y, src, axis=0), dst,
                               num_segments=_N)

    b2p = jnp.pad(b2, (0, _D_PAD - _D_OUT))
    out = _fin(agg2, norm_dst, b2p[None, :])
    return out[:, :_D_OUT]


# speedup vs baseline: 5.6341x; 5.4057x over previous
import jax
import jax.numpy as jnp
from jax import lax
from jax.experimental import pallas as pl
from jax.experimental.pallas import tpu as pltpu
from jax.experimental.pallas import tpu_sc as plsc

_N = 10000
_E = 320000
_D_IN = 128
_D_HID = 128
_D_OUT = 40
_D_PAD = 48

_NP = 10240
_C = 128
_EP = 327680
_NBUF = 4
_RPS = _NP // 16

_ROWS = 1024


def _mesh():
    return plsc.VectorSubcoreMesh(core_axis_name="c", subcore_axis_name="s",
                                  num_cores=2, num_subcores=16)


def _mm1_body(x_ref, ns_ref, w_ref, o_ref):
    o_ref[...] = jnp.dot(x_ref[...] * ns_ref[...], w_ref[...],
                         preferred_element_type=jnp.float32)


def _mm1(xp, norm_src, W1):
    return pl.pallas_call(
        _mm1_body,
        grid=(_NP // _ROWS,),
        in_specs=[
            pl.BlockSpec((_ROWS, _D_IN), lambda i: (i, 0)),
            pl.BlockSpec((_ROWS, 1), lambda i: (i, 0)),
            pl.BlockSpec((_D_IN, _D_HID), lambda i: (0, 0)),
        ],
        out_specs=pl.BlockSpec((_ROWS, _D_HID), lambda i: (i, 0)),
        out_shape=jax.ShapeDtypeStruct((_NP, _D_HID), jnp.float32),
    )(xp, norm_src, W1)


def _mm2_body(p0_ref, p1_ref, nd_ref, ns_ref, b1_ref, w_ref, o_ref):
    agg = jnp.concatenate([p0_ref[0], p1_ref[0]], axis=1)
    h = jax.nn.relu(agg * nd_ref[...] + b1_ref[...])
    o_ref[...] = jnp.dot(h * ns_ref[...], w_ref[...],
                         preferred_element_type=jnp.float32)


def _mm2(agg1, norm_dst, norm_src, b1, W2p):
    return pl.pallas_call(
        _mm2_body,
        grid=(_NP // _ROWS,),
        in_specs=[
            pl.BlockSpec((1, _ROWS, _D_HID // 2), lambda i: (0, i, 0)),
            pl.BlockSpec((1, _ROWS, _D_HID // 2), lambda i: (1, i, 0)),
            pl.BlockSpec((_ROWS, 1), lambda i: (i, 0)),
            pl.BlockSpec((_ROWS, 1), lambda i: (i, 0)),
            pl.BlockSpec((1, _D_HID), lambda i: (0, 0)),
            pl.BlockSpec((_D_HID, _D_PAD), lambda i: (0, 0)),
        ],
        out_specs=pl.BlockSpec((_ROWS, _D_PAD), lambda i: (i, 0)),
        out_shape=jax.ShapeDtypeStruct((_NP, _D_PAD), jnp.float32),
    )(agg1, agg1, norm_dst, norm_src, b1, W2p)


def _fin_body(p0_ref, p1_ref, nd_ref, b2_ref, o_ref):
    o_ref[...] = (p0_ref[...] + p1_ref[...]) * nd_ref[...] + b2_ref[...]


def _fin(p0, p1, norm_dst, b2p):
    return pl.pallas_call(
        _fin_body,
        grid=(_NP // _ROWS,),
        in_specs=[
            pl.BlockSpec((_ROWS, _D_PAD), lambda i: (i, 0)),
            pl.BlockSpec((_ROWS, _D_PAD), lambda i: (i, 0)),
            pl.BlockSpec((_ROWS, 1), lambda i: (i, 0)),
            pl.BlockSpec((1, _D_PAD), lambda i: (0, 0)),
        ],
        out_specs=pl.BlockSpec((_ROWS, _D_PAD), lambda i: (i, 0)),
        out_shape=jax.ShapeDtypeStruct((_NP, _D_PAD), jnp.float32),
    )(p0, p1, norm_dst, b2p)


def _zero_tile_buf(zb, d):
    z = jnp.zeros((16,), jnp.float32)
    for r in range(16):
        for k in range(d // 16):
            zb[r, pl.ds(k * 16, 16)] = z


def _zero_acc_slice(zb, acc, base):
    def body(i, carry):
        pltpu.sync_copy(zb, acc.at[pl.ds(base + i * 16, 16)])
        return carry
    lax.fori_loop(0, _RPS // 16, body, 0)


def _spmm_body(table_h, src_h, dst_h, sidx, didx, bufs, acc, gsems, ssems,
               w, nch):
    pltpu.sync_copy(src_h.at[w], sidx)
    pltpu.sync_copy(dst_h.at[w], didx)

    def gather(j, b):
        return pltpu.async_copy(table_h.at[sidx.at[j]], bufs[b], gsems[b])

    def gwait(j, b):
        pltpu.make_async_copy(table_h.at[sidx.at[j]], bufs[b],
                              gsems[b]).wait()

    def scatter(j, b):
        return pltpu.async_copy(bufs[b], acc.at[didx.at[j]], ssems[b],
                                add=True)

    for b in range(_NBUF):
        gather(b, b)

    def group(g, carry):
        for b in range(_NBUF):
            j = g * _NBUF + b
            gwait(j, b)
            scatter(j, b).wait()
            nj = j + _NBUF

            @pl.when(nj < nch)
            def _():
                gather(nj, b)
        return carry

    lax.fori_loop(0, nch // _NBUF, group, 0)


def _spmm1_sc():
    nch = _EP // 16 // _C

    def wrapped(table_h, src_h, dst_h, out_h, sidx, didx, b0, b1, b2, b3, zb,
                acc, g0, g1, g2, g3, s0, s1, s2, s3):
        c = lax.axis_index("c")
        s = lax.axis_index("s")
        w = c * 16 + s
        _zero_tile_buf(zb, 64)
        _zero_acc_slice(zb, acc, s * _RPS)
        plsc.subcore_barrier()
        _spmm_body(table_h, src_h, dst_h, sidx, didx, (b0, b1, b2, b3), acc,
                   (g0, g1, g2, g3), (s0, s1, s2, s3), w, nch)
        plsc.subcore_barrier()
        pltpu.sync_copy(acc.at[pl.ds(s * _RPS, _RPS)],
                        out_h.at[c, pl.ds(s * _RPS, _RPS)])

    return pl.kernel(
        wrapped,
        out_type=jax.ShapeDtypeStruct((2, _NP, 64), jnp.float32),
        mesh=_mesh(),
        compiler_params=pltpu.CompilerParams(use_tc_tiling_on_sc=False),
        scratch_types=[
            pltpu.VMEM((nch, _C), jnp.int32),
            pltpu.VMEM((nch, _C), jnp.int32),
            *[pltpu.VMEM((_C, 64), jnp.float32) for _ in range(_NBUF)],
            pltpu.VMEM((16, 64), jnp.float32),
            pltpu.VMEM_SHARED((_NP, 64), jnp.float32),
            *[pltpu.SemaphoreType.DMA for _ in range(2 * _NBUF)],
        ],
    )


def _spmm2_sc():
    nch = _EP // 32 // _C

    def wrapped(table_h, src_h, dst_h, out_h, sidx, didx, b0, b1, b2, b3, zb,
                acc, g0, g1, g2, g3, s0, s1, s2, s3):
        c = lax.axis_index("c")
        s = lax.axis_index("s")
        w = c * 16 + s
        _zero_tile_buf(zb, _D_PAD)
        _zero_acc_slice(zb, acc, s * _RPS)
        plsc.subcore_barrier()
        _spmm_body(table_h, src_h, dst_h, sidx, didx, (b0, b1, b2, b3), acc,
                   (g0, g1, g2, g3), (s0, s1, s2, s3), w, nch)
        plsc.subcore_barrier()
        pltpu.sync_copy(acc.at[pl.ds(s * _RPS, _RPS)],
                        out_h.at[c, pl.ds(s * _RPS, _RPS)])

    return pl.kernel(
        wrapped,
        out_type=jax.ShapeDtypeStruct((2, _NP, _D_PAD), jnp.float32),
        mesh=_mesh(),
        compiler_params=pltpu.CompilerParams(use_tc_tiling_on_sc=False),
        scratch_types=[
            pltpu.VMEM((nch, _C), jnp.int32),
            pltpu.VMEM((nch, _C), jnp.int32),
            *[pltpu.VMEM((_C, _D_PAD), jnp.float32) for _ in range(_NBUF)],
            pltpu.VMEM((16, _D_PAD), jnp.float32),
            pltpu.VMEM_SHARED((_NP, _D_PAD), jnp.float32),
            *[pltpu.SemaphoreType.DMA for _ in range(2 * _NBUF)],
        ],
    )


def _deg_sc():
    nch = _EP // 32 // _C

    def body(src_h, dst_h, out_h, sidx, didx, obs, obd, zb, acc):
        c = lax.axis_index("c")
        s = lax.axis_index("s")
        w = c * 16 + s

        _zero_tile_buf(zb, 16)
        lane = lax.iota(jnp.int32, 16)
        e0 = jnp.where(lane == 0, 1.0, 0.0)
        e1 = jnp.where(lane == 1, 1.0, 0.0)
        for r in range(_C):
            obs[r, pl.ds(0, 16)] = e0
            obd[r, pl.ds(0, 16)] = e1
        _zero_acc_slice(zb, acc, s * _RPS)

        pltpu.sync_copy(src_h.at[w], sidx)
        pltpu.sync_copy(dst_h.at[w], didx)
        plsc.subcore_barrier()

        def chunk(j, carry):
            pltpu.sync_copy(obs, acc.at[sidx.at[j]], add=True)
            pltpu.sync_copy(obd, acc.at[didx.at[j]], add=True)
            return carry

        lax.fori_loop(0, nch, chunk, 0)

        plsc.subcore_barrier()
        sl = pl.ds(s * _RPS, _RPS)
        pltpu.sync_copy(acc.at[sl], out_h.at[c, sl])

    return pl.kernel(
        body,
        out_type=jax.ShapeDtypeStruct((2, _NP, 16), jnp.float32),
        mesh=_mesh(),
        compiler_params=pltpu.CompilerParams(use_tc_tiling_on_sc=False),
        scratch_types=[
            pltpu.VMEM((nch, _C), jnp.int32),
            pltpu.VMEM((nch, _C), jnp.int32),
            pltpu.VMEM((_C, 16), jnp.float32),
            pltpu.VMEM((_C, 16), jnp.float32),
            pltpu.VMEM((16, 16), jnp.float32),
            pltpu.VMEM_SHARED((_NP, 16), jnp.float32),
        ],
    )


def kernel(x, edge_index, W1, b1, W2, b2):
    src = edge_index[0]
    dst = edge_index[1]
    srcp = jnp.pad(src, (0, _EP - _E), constant_values=_N)
    dstp = jnp.pad(dst, (0, _EP - _E), constant_values=_N)
    src32 = srcp.reshape(32, _EP // 32 // _C, _C)
    dst32 = dstp.reshape(32, _EP // 32 // _C, _C)
    s16 = srcp.reshape(16, _EP // 16 // _C, _C)
    d16 = dstp.reshape(16, _EP // 16 // _C, _C)
    srcA = jnp.concatenate([2 * s16, 2 * s16 + 1], axis=0)
    dstA = jnp.concatenate([d16, d16], axis=0)

    degp = _deg_sc()(src32, dst32)
    deg = degp[0] + degp[1]
    deg_out = deg[:, 0:1]
    deg_in = deg[:, 1:2]
    norm_src = jnp.where(deg_out > 0, deg_out**-0.5, 0.0)
    norm_dst = jnp.where(deg_in > 0, deg_in**-0.5, 0.0)

    xp = jnp.pad(x, ((0, _NP - _N), (0, 0)))
    t1 = _mm1(xp, norm_src, W1)
    agg1 = _spmm1_sc()(t1.reshape(2 * _NP, 64), srcA, dstA)

    W2p = jnp.pad(W2, ((0, 0), (0, _D_PAD - _D_OUT)))
    t2 = _mm2(agg1, norm_dst, norm_src, b1[None, :], W2p)
    agg2 = _spmm2_sc()(t2, src32, dst32)

    b2p = jnp.pad(b2, (0, _D_PAD - _D_OUT))
    out = _fin(agg2[0], agg2[1], norm_dst, b2p[None, :])
    return out[:_N, :_D_OUT]


# speedup vs baseline: 12.7139x; 2.2566x over previous
import jax
import jax.numpy as jnp
from jax import lax
from jax.experimental import pallas as pl
from jax.experimental.pallas import tpu as pltpu
from jax.experimental.pallas import tpu_sc as plsc

_N = 10000
_E = 320000
_D_IN = 128
_D_HID = 128
_D_OUT = 40
_D_PAD = 48

_NP = 10240
_C = 128
_EP = 327680
_NBUF = 4
_RPS = _NP // 16

_ROWS = 1024


def _mesh():
    return plsc.VectorSubcoreMesh(core_axis_name="c", subcore_axis_name="s",
                                  num_cores=2, num_subcores=16)


def _mm1_body(x_ref, ns_ref, w_ref, o_ref):
    o_ref[...] = jnp.dot(x_ref[...] * ns_ref[...], w_ref[...],
                         preferred_element_type=jnp.float32)


def _mm1(xp, norm_src, W1):
    return pl.pallas_call(
        _mm1_body,
        grid=(_NP // _ROWS,),
        in_specs=[
            pl.BlockSpec((_ROWS, _D_IN), lambda i: (i, 0)),
            pl.BlockSpec((_ROWS, 1), lambda i: (i, 0)),
            pl.BlockSpec((_D_IN, _D_HID), lambda i: (0, 0)),
        ],
        out_specs=pl.BlockSpec((_ROWS, _D_HID), lambda i: (i, 0)),
        out_shape=jax.ShapeDtypeStruct((_NP, _D_HID), jnp.float32),
    )(xp, norm_src, W1)


def _mm2_body(p0_ref, p1_ref, nd_ref, ns_ref, b1_ref, w_ref, o_ref):
    agg = jnp.concatenate([p0_ref[0], p1_ref[0]], axis=1)
    h = jax.nn.relu(agg * nd_ref[...] + b1_ref[...])
    o_ref[...] = jnp.dot(h * ns_ref[...], w_ref[...],
                         preferred_element_type=jnp.float32)


def _mm2(agg1, norm_dst, norm_src, b1, W2p):
    return pl.pallas_call(
        _mm2_body,
        grid=(_NP // _ROWS,),
        in_specs=[
            pl.BlockSpec((1, _ROWS, _D_HID // 2), lambda i: (0, i, 0)),
            pl.BlockSpec((1, _ROWS, _D_HID // 2), lambda i: (1, i, 0)),
            pl.BlockSpec((_ROWS, 1), lambda i: (i, 0)),
            pl.BlockSpec((_ROWS, 1), lambda i: (i, 0)),
            pl.BlockSpec((1, _D_HID), lambda i: (0, 0)),
            pl.BlockSpec((_D_HID, _D_PAD), lambda i: (0, 0)),
        ],
        out_specs=pl.BlockSpec((_ROWS, _D_PAD), lambda i: (i, 0)),
        out_shape=jax.ShapeDtypeStruct((_NP, _D_PAD), jnp.float32),
    )(agg1, agg1, norm_dst, norm_src, b1, W2p)


def _fin_body(p0_ref, p1_ref, nd_ref, b2_ref, o_ref):
    o_ref[...] = (p0_ref[...] + p1_ref[...]) * nd_ref[...] + b2_ref[...]


def _fin(p0, p1, norm_dst, b2p):
    return pl.pallas_call(
        _fin_body,
        grid=(_NP // _ROWS,),
        in_specs=[
            pl.BlockSpec((_ROWS, _D_PAD), lambda i: (i, 0)),
            pl.BlockSpec((_ROWS, _D_PAD), lambda i: (i, 0)),
            pl.BlockSpec((_ROWS, 1), lambda i: (i, 0)),
            pl.BlockSpec((1, _D_PAD), lambda i: (0, 0)),
        ],
        out_specs=pl.BlockSpec((_ROWS, _D_PAD), lambda i: (i, 0)),
        out_shape=jax.ShapeDtypeStruct((_NP, _D_PAD), jnp.float32),
    )(p0, p1, norm_dst, b2p)


def _zero_tile_buf(zb, d):
    z = jnp.zeros((16,), jnp.float32)
    for r in range(16):
        for k in range(d // 16):
            zb[r, pl.ds(k * 16, 16)] = z


def _zero_acc_slice(zb, acc, base):
    def body(i, carry):
        pltpu.sync_copy(zb, acc.at[pl.ds(base + i * 16, 16)])
        return carry
    lax.fori_loop(0, _RPS // 16, body, 0)


def _spmm_body(table_h, src_h, dst_h, sidx, didx, bufs, acc, gsems, ssems,
               w, nch):
    pltpu.sync_copy(src_h.at[w], sidx)
    pltpu.sync_copy(dst_h.at[w], didx)

    def gather(j, b):
        return pltpu.async_copy(table_h.at[sidx.at[j]], bufs[b], gsems[b])

    def gwait(j, b):
        pltpu.make_async_copy(table_h.at[sidx.at[j]], bufs[b],
                              gsems[b]).wait()

    def scatter(j, b):
        return pltpu.async_copy(bufs[b], acc.at[didx.at[j]], ssems[b],
                                add=True)

    def swait(j, b):
        pltpu.make_async_copy(bufs[b], acc.at[didx.at[j]], ssems[b]).wait()

    for b in range(_NBUF):
        gather(b, b)

    def group(g, carry):
        for b in range(_NBUF):
            j = g * _NBUF + b
            gwait(j, b)
            scatter(j, b)
        for b in range(_NBUF):
            j = g * _NBUF + b
            swait(j, b)
            nj = j + _NBUF

            @pl.when(nj < nch)
            def _():
                gather(nj, b)
        return carry

    lax.fori_loop(0, nch // _NBUF, group, 0)


def _spmm1_sc():
    nch = _EP // 16 // _C

    def wrapped(table_h, src_h, dst_h, out_h, sidx, didx, b0, b1, b2, b3, zb,
                acc, g0, g1, g2, g3, s0, s1, s2, s3):
        c = lax.axis_index("c")
        s = lax.axis_index("s")
        w = c * 16 + s
        _zero_tile_buf(zb, 64)
        _zero_acc_slice(zb, acc, s * _RPS)
        plsc.subcore_barrier()
        _spmm_body(table_h, src_h, dst_h, sidx, didx, (b0, b1, b2, b3), acc,
                   (g0, g1, g2, g3), (s0, s1, s2, s3), w, nch)
        plsc.subcore_barrier()
        pltpu.sync_copy(acc.at[pl.ds(s * _RPS, _RPS)],
                        out_h.at[c, pl.ds(s * _RPS, _RPS)])

    return pl.kernel(
        wrapped,
        out_type=jax.ShapeDtypeStruct((2, _NP, 64), jnp.float32),
        mesh=_mesh(),
        compiler_params=pltpu.CompilerParams(use_tc_tiling_on_sc=False),
        scratch_types=[
            pltpu.VMEM((nch, _C), jnp.int32),
            pltpu.VMEM((nch, _C), jnp.int32),
            *[pltpu.VMEM((_C, 64), jnp.float32) for _ in range(_NBUF)],
            pltpu.VMEM((16, 64), jnp.float32),
            pltpu.VMEM_SHARED((_NP, 64), jnp.float32),
            *[pltpu.SemaphoreType.DMA for _ in range(2 * _NBUF)],
        ],
    )


def _spmm2_sc():
    nch = _EP // 32 // _C

    def wrapped(table_h, src_h, dst_h, out_h, sidx, didx, b0, b1, b2, b3, zb,
                acc, g0, g1, g2, g3, s0, s1, s2, s3):
        c = lax.axis_index("c")
        s = lax.axis_index("s")
        w = c * 16 + s
        _zero_tile_buf(zb, _D_PAD)
        _zero_acc_slice(zb, acc, s * _RPS)
        plsc.subcore_barrier()
        _spmm_body(table_h, src_h, dst_h, sidx, didx, (b0, b1, b2, b3), acc,
                   (g0, g1, g2, g3), (s0, s1, s2, s3), w, nch)
        plsc.subcore_barrier()
        pltpu.sync_copy(acc.at[pl.ds(s * _RPS, _RPS)],
                        out_h.at[c, pl.ds(s * _RPS, _RPS)])

    return pl.kernel(
        wrapped,
        out_type=jax.ShapeDtypeStruct((2, _NP, _D_PAD), jnp.float32),
        mesh=_mesh(),
        compiler_params=pltpu.CompilerParams(use_tc_tiling_on_sc=False),
        scratch_types=[
            pltpu.VMEM((nch, _C), jnp.int32),
            pltpu.VMEM((nch, _C), jnp.int32),
            *[pltpu.VMEM((_C, _D_PAD), jnp.float32) for _ in range(_NBUF)],
            pltpu.VMEM((16, _D_PAD), jnp.float32),
            pltpu.VMEM_SHARED((_NP, _D_PAD), jnp.float32),
            *[pltpu.SemaphoreType.DMA for _ in range(2 * _NBUF)],
        ],
    )


def _deg_sc():
    nch = _EP // 32 // _C

    def body(src_h, dst_h, out_h, sidx, didx, obs, obd, zb, acc):
        c = lax.axis_index("c")
        s = lax.axis_index("s")
        w = c * 16 + s

        _zero_tile_buf(zb, 16)
        lane = lax.iota(jnp.int32, 16)
        e0 = jnp.where(lane == 0, 1.0, 0.0)
        e1 = jnp.where(lane == 1, 1.0, 0.0)
        for r in range(_C):
            obs[r, pl.ds(0, 16)] = e0
            obd[r, pl.ds(0, 16)] = e1
        _zero_acc_slice(zb, acc, s * _RPS)

        pltpu.sync_copy(src_h.at[w], sidx)
        pltpu.sync_copy(dst_h.at[w], didx)
        plsc.subcore_barrier()

        def chunk(j, carry):
            pltpu.sync_copy(obs, acc.at[sidx.at[j]], add=True)
            pltpu.sync_copy(obd, acc.at[didx.at[j]], add=True)
            return carry

        lax.fori_loop(0, nch, chunk, 0)

        plsc.subcore_barrier()
        sl = pl.ds(s * _RPS, _RPS)
        pltpu.sync_copy(acc.at[sl], out_h.at[c, sl])

    return pl.kernel(
        body,
        out_type=jax.ShapeDtypeStruct((2, _NP, 16), jnp.float32),
        mesh=_mesh(),
        compiler_params=pltpu.CompilerParams(use_tc_tiling_on_sc=False),
        scratch_types=[
            pltpu.VMEM((nch, _C), jnp.int32),
            pltpu.VMEM((nch, _C), jnp.int32),
            pltpu.VMEM((_C, 16), jnp.float32),
            pltpu.VMEM((_C, 16), jnp.float32),
            pltpu.VMEM((16, 16), jnp.float32),
            pltpu.VMEM_SHARED((_NP, 16), jnp.float32),
        ],
    )


def kernel(x, edge_index, W1, b1, W2, b2):
    src = edge_index[0]
    dst = edge_index[1]
    pad_rows = _N + jnp.arange(_EP - _E, dtype=jnp.int32) % (_NP - _N)
    srcp = jnp.concatenate([src, pad_rows])
    dstp = jnp.concatenate([dst, pad_rows])
    src32 = srcp.reshape(32, _EP // 32 // _C, _C)
    dst32 = dstp.reshape(32, _EP // 32 // _C, _C)
    s16 = srcp.reshape(16, _EP // 16 // _C, _C)
    d16 = dstp.reshape(16, _EP // 16 // _C, _C)
    srcA = jnp.concatenate([2 * s16, 2 * s16 + 1], axis=0)
    dstA = jnp.concatenate([d16, d16], axis=0)

    degp = _deg_sc()(src32, dst32)
    deg = degp[0] + degp[1]
    deg_out = deg[:, 0:1]
    deg_in = deg[:, 1:2]
    norm_src = jnp.where(deg_out > 0, deg_out**-0.5, 0.0)
    norm_dst = jnp.where(deg_in > 0, deg_in**-0.5, 0.0)

    xp = jnp.pad(x, ((0, _NP - _N), (0, 0)))
    t1 = _mm1(xp, norm_src, W1)
    agg1 = _spmm1_sc()(t1.reshape(2 * _NP, 64), srcA, dstA)

    W2p = jnp.pad(W2, ((0, 0), (0, _D_PAD - _D_OUT)))
    t2 = _mm2(agg1, norm_dst, norm_src, b1[None, :], W2p)
    agg2 = _spmm2_sc()(t2, src32, dst32)

    b2p = jnp.pad(b2, (0, _D_PAD - _D_OUT))
    out = _fin(agg2[0], agg2[1], norm_dst, b2p[None, :])
    return out[:_N, :_D_OUT]


# speedup vs baseline: 13.7389x; 1.0806x over previous
import jax
import jax.numpy as jnp
from jax import lax
from jax.experimental import pallas as pl
from jax.experimental.pallas import tpu as pltpu
from jax.experimental.pallas import tpu_sc as plsc

_N = 10000
_E = 320000
_D_IN = 128
_D_HID = 128
_D_OUT = 40
_D_PAD = 48

_NP = 10240
_C = 128
_EP = 327680
_NBUF = 5
_RPS = _NP // 16

_ROWS = 1024


def _mesh():
    return plsc.VectorSubcoreMesh(core_axis_name="c", subcore_axis_name="s",
                                  num_cores=2, num_subcores=16)


def _norms(dp_ref):
    d = dp_ref[0] + dp_ref[1]
    ns = d[:, 0:1]
    nd = d[:, 1:2]
    ns = jnp.where(ns > 0, lax.rsqrt(ns), 0.0)
    nd = jnp.where(nd > 0, lax.rsqrt(nd), 0.0)
    return ns, nd


def _mm1_body(x_ref, dp_ref, w_ref, o_ref):
    ns, _ = _norms(dp_ref)
    o_ref[...] = jnp.dot(x_ref[...] * ns, w_ref[...],
                         preferred_element_type=jnp.float32)


def _mm1(xp, degp, W1):
    return pl.pallas_call(
        _mm1_body,
        grid=(_NP // _ROWS,),
        in_specs=[
            pl.BlockSpec((_ROWS, _D_IN), lambda i: (i, 0)),
            pl.BlockSpec((2, _ROWS, 16), lambda i: (0, i, 0)),
            pl.BlockSpec((_D_IN, _D_HID), lambda i: (0, 0)),
        ],
        out_specs=pl.BlockSpec((_ROWS, _D_HID), lambda i: (i, 0)),
        out_shape=jax.ShapeDtypeStruct((_NP, _D_HID), jnp.float32),
    )(xp, degp, W1)


def _mm2_body(p0_ref, p1_ref, dp_ref, b1_ref, w_ref, o_ref):
    ns, nd = _norms(dp_ref)
    agg = jnp.concatenate([p0_ref[0], p1_ref[0]], axis=1)
    h = jax.nn.relu(agg * nd + b1_ref[...])
    o_ref[...] = jnp.dot(h * ns, w_ref[...],
                         preferred_element_type=jnp.float32)


def _mm2(agg1, degp, b1, W2p):
    return pl.pallas_call(
        _mm2_body,
        grid=(_NP // _ROWS,),
        in_specs=[
            pl.BlockSpec((1, _ROWS, _D_HID // 2), lambda i: (0, i, 0)),
            pl.BlockSpec((1, _ROWS, _D_HID // 2), lambda i: (1, i, 0)),
            pl.BlockSpec((2, _ROWS, 16), lambda i: (0, i, 0)),
            pl.BlockSpec((1, _D_HID), lambda i: (0, 0)),
            pl.BlockSpec((_D_HID, _D_PAD), lambda i: (0, 0)),
        ],
        out_specs=pl.BlockSpec((_ROWS, _D_PAD), lambda i: (i, 0)),
        out_shape=jax.ShapeDtypeStruct((_NP, _D_PAD), jnp.float32),
    )(agg1, agg1, degp, b1, W2p)


def _fin_body(p0_ref, p1_ref, dp_ref, b2_ref, o_ref):
    _, nd = _norms(dp_ref)
    o_ref[...] = (p0_ref[0] + p1_ref[0]) * nd + b2_ref[...]


def _fin(agg2, degp, b2p):
    return pl.pallas_call(
        _fin_body,
        grid=(_NP // _ROWS,),
        in_specs=[
            pl.BlockSpec((1, _ROWS, _D_PAD), lambda i: (0, i, 0)),
            pl.BlockSpec((1, _ROWS, _D_PAD), lambda i: (1, i, 0)),
            pl.BlockSpec((2, _ROWS, 16), lambda i: (0, i, 0)),
            pl.BlockSpec((1, _D_PAD), lambda i: (0, 0)),
        ],
        out_specs=pl.BlockSpec((_ROWS, _D_PAD), lambda i: (i, 0)),
        out_shape=jax.ShapeDtypeStruct((_NP, _D_PAD), jnp.float32),
    )(agg2, agg2, degp, b2p)


def _zero_tile_buf(zb, d):
    z = jnp.zeros((16,), jnp.float32)
    for r in range(16):
        for k in range(d // 16):
            zb[r, pl.ds(k * 16, 16)] = z


def _zero_acc_slice(zb, acc, base):
    def body(i, carry):
        pltpu.sync_copy(zb, acc.at[pl.ds(base + i * 16, 16)])
        return carry
    lax.fori_loop(0, _RPS // 16, body, 0)


def _spmm_body(table_h, src_h, dst_h, sidx, didx, bufs, acc, gsems, ssems,
               w, nch):
    pltpu.sync_copy(src_h.at[w], sidx)
    pltpu.sync_copy(dst_h.at[w], didx)

    def gather(j, b):
        return pltpu.async_copy(table_h.at[sidx.at[j]], bufs[b], gsems[b])

    def gwait(j, b):
        pltpu.make_async_copy(table_h.at[sidx.at[j]], bufs[b],
                              gsems[b]).wait()

    def scatter(j, b):
        return pltpu.async_copy(bufs[b], acc.at[didx.at[j]], ssems[b],
                                add=True)

    def swait(j, b):
        pltpu.make_async_copy(bufs[b], acc.at[didx.at[j]], ssems[b]).wait()

    for b in range(_NBUF):
        gather(b, b)

    def group(g, carry):
        for b in range(_NBUF):
            j = g * _NBUF + b
            gwait(j, b)
            scatter(j, b)
        for b in range(_NBUF):
            j = g * _NBUF + b
            swait(j, b)
            nj = j + _NBUF

            @pl.when(nj < nch)
            def _():
                gather(nj, b)
        return carry

    lax.fori_loop(0, nch // _NBUF, group, 0)


def _spmm1_sc():
    nch = _EP // 16 // _C

    def wrapped(table_h, src_h, dst_h, out_h, sidx, didx, *rest):
        bufs, zb, acc = rest[:_NBUF], rest[_NBUF], rest[_NBUF + 1]
        gsems = rest[_NBUF + 2:2 * _NBUF + 2]
        ssems = rest[2 * _NBUF + 2:]
        c = lax.axis_index("c")
        s = lax.axis_index("s")
        w = c * 16 + s
        _zero_tile_buf(zb, 64)
        _zero_acc_slice(zb, acc, s * _RPS)
        plsc.subcore_barrier()
        _spmm_body(table_h, src_h, dst_h, sidx, didx, bufs, acc,
                   gsems, ssems, w, nch)
        plsc.subcore_barrier()
        pltpu.sync_copy(acc.at[pl.ds(s * _RPS, _RPS)],
                        out_h.at[c, pl.ds(s * _RPS, _RPS)])

    return pl.kernel(
        wrapped,
        out_type=jax.ShapeDtypeStruct((2, _NP, 64), jnp.float32),
        mesh=_mesh(),
        compiler_params=pltpu.CompilerParams(use_tc_tiling_on_sc=False),
        scratch_types=[
            pltpu.VMEM((nch, _C), jnp.int32),
            pltpu.VMEM((nch, _C), jnp.int32),
            *[pltpu.VMEM((_C, 64), jnp.float32) for _ in range(_NBUF)],
            pltpu.VMEM((16, 64), jnp.float32),
            pltpu.VMEM_SHARED((_NP, 64), jnp.float32),
            *[pltpu.SemaphoreType.DMA for _ in range(2 * _NBUF)],
        ],
    )


def _spmm2_sc():
    nch = _EP // 32 // _C

    def wrapped(table_h, src_h, dst_h, out_h, sidx, didx, *rest):
        bufs, zb, acc = rest[:_NBUF], rest[_NBUF], rest[_NBUF + 1]
        gsems = rest[_NBUF + 2:2 * _NBUF + 2]
        ssems = rest[2 * _NBUF + 2:]
        c = lax.axis_index("c")
        s = lax.axis_index("s")
        w = c * 16 + s
        _zero_tile_buf(zb, _D_PAD)
        _zero_acc_slice(zb, acc, s * _RPS)
        plsc.subcore_barrier()
        _spmm_body(table_h, src_h, dst_h, sidx, didx, bufs, acc,
                   gsems, ssems, w, nch)
        plsc.subcore_barrier()
        pltpu.sync_copy(acc.at[pl.ds(s * _RPS, _RPS)],
                        out_h.at[c, pl.ds(s * _RPS, _RPS)])

    return pl.kernel(
        wrapped,
        out_type=jax.ShapeDtypeStruct((2, _NP, _D_PAD), jnp.float32),
        mesh=_mesh(),
        compiler_params=pltpu.CompilerParams(use_tc_tiling_on_sc=False),
        scratch_types=[
            pltpu.VMEM((nch, _C), jnp.int32),
            pltpu.VMEM((nch, _C), jnp.int32),
            *[pltpu.VMEM((_C, _D_PAD), jnp.float32) for _ in range(_NBUF)],
            pltpu.VMEM((16, _D_PAD), jnp.float32),
            pltpu.VMEM_SHARED((_NP, _D_PAD), jnp.float32),
            *[pltpu.SemaphoreType.DMA for _ in range(2 * _NBUF)],
        ],
    )


def _deg_sc():
    nch = _EP // 32 // _C

    def body(src_h, dst_h, out_h, sidx, didx, obs, obd, zb, acc, *sems):
        c = lax.axis_index("c")
        s = lax.axis_index("s")
        w = c * 16 + s

        _zero_tile_buf(zb, 16)
        lane = lax.iota(jnp.int32, 16)
        e0 = jnp.where(lane == 0, 1.0, 0.0)
        e1 = jnp.where(lane == 1, 1.0, 0.0)
        for r in range(_C):
            obs[r, pl.ds(0, 16)] = e0
            obd[r, pl.ds(0, 16)] = e1
        _zero_acc_slice(zb, acc, s * _RPS)

        pltpu.sync_copy(src_h.at[w], sidx)
        pltpu.sync_copy(dst_h.at[w], didx)
        plsc.subcore_barrier()

        def chunk(j, carry):
            @pl.when(j > 0)
            def _():
                pltpu.make_async_copy(obs, acc.at[sidx.at[j - 1]],
                                      sems[0]).wait()
                pltpu.make_async_copy(obd, acc.at[didx.at[j - 1]],
                                      sems[1]).wait()
            pltpu.async_copy(obs, acc.at[sidx.at[j]], sems[0], add=True)
            pltpu.async_copy(obd, acc.at[didx.at[j]], sems[1], add=True)
            return carry

        lax.fori_loop(0, nch, chunk, 0)
        pltpu.make_async_copy(obs, acc.at[sidx.at[nch - 1]], sems[0]).wait()
        pltpu.make_async_copy(obd, acc.at[didx.at[nch - 1]], sems[1]).wait()

        plsc.subcore_barrier()
        sl = pl.ds(s * _RPS, _RPS)
        pltpu.sync_copy(acc.at[sl], out_h.at[c, sl])

    return pl.kernel(
        body,
        out_type=jax.ShapeDtypeStruct((2, _NP, 16), jnp.float32),
        mesh=_mesh(),
        compiler_params=pltpu.CompilerParams(use_tc_tiling_on_sc=False),
        scratch_types=[
            pltpu.VMEM((nch, _C), jnp.int32),
            pltpu.VMEM((nch, _C), jnp.int32),
            pltpu.VMEM((_C, 16), jnp.float32),
            pltpu.VMEM((_C, 16), jnp.float32),
            pltpu.VMEM((16, 16), jnp.float32),
            pltpu.VMEM_SHARED((_NP, 16), jnp.float32),
            pltpu.SemaphoreType.DMA,
            pltpu.SemaphoreType.DMA,
        ],
    )


def kernel(x, edge_index, W1, b1, W2, b2):
    src = edge_index[0]
    dst = edge_index[1]
    pad_rows = _N + jnp.arange(_EP - _E, dtype=jnp.int32) % (_NP - _N)
    srcp = jnp.concatenate([src, pad_rows])
    dstp = jnp.concatenate([dst, pad_rows])
    src32 = srcp.reshape(32, _EP // 32 // _C, _C)
    dst32 = dstp.reshape(32, _EP // 32 // _C, _C)
    s16 = srcp.reshape(16, _EP // 16 // _C, _C)
    d16 = dstp.reshape(16, _EP // 16 // _C, _C)
    srcA = jnp.concatenate([2 * s16, 2 * s16 + 1], axis=0)
    dstA = jnp.concatenate([d16, d16], axis=0)

    degp = _deg_sc()(src32, dst32)

    xp = jnp.pad(x, ((0, _NP - _N), (0, 0)))
    t1 = _mm1(xp, degp, W1)
    agg1 = _spmm1_sc()(t1.reshape(2 * _NP, 64), srcA, dstA)

    W2p = jnp.pad(W2, ((0, 0), (0, _D_PAD - _D_OUT)))
    t2 = _mm2(agg1, degp, b1[None, :], W2p)
    agg2 = _spmm2_sc()(t2, src32, dst32)

    b2p = jnp.pad(b2, (0, _D_PAD - _D_OUT))
    out = _fin(agg2, degp, b2p[None, :])
    return out[:_N, :_D_OUT]


# speedup vs baseline: 13.7685x; 1.0022x over previous
import jax
import jax.numpy as jnp
from jax import lax
from jax.experimental import pallas as pl
from jax.experimental.pallas import tpu as pltpu
from jax.experimental.pallas import tpu_sc as plsc

_N = 10000
_E = 320000
_D_IN = 128
_D_HID = 128
_D_OUT = 40
_D_PAD = 48

_NP = 10240
_C = 128
_EP = 327680
_NBUF = 5
_RPS = _NP // 16

_ROWS = 1024


def _mesh():
    return plsc.VectorSubcoreMesh(core_axis_name="c", subcore_axis_name="s",
                                  num_cores=2, num_subcores=16)


def _norms(dp_ref):
    d = dp_ref[0] + dp_ref[1]
    ns = d[:, 0:1]
    nd = d[:, 1:2]
    ns = jnp.where(ns > 0, lax.rsqrt(ns), 0.0)
    nd = jnp.where(nd > 0, lax.rsqrt(nd), 0.0)
    return ns, nd


def _mm1_body(x_ref, dp_ref, w_ref, o_ref):
    ns, _ = _norms(dp_ref)
    o_ref[...] = jnp.dot(x_ref[...] * ns, w_ref[...],
                         preferred_element_type=jnp.float32)


def _mm1(xp, degp, W1):
    return pl.pallas_call(
        _mm1_body,
        grid=(_NP // _ROWS,),
        in_specs=[
            pl.BlockSpec((_ROWS, _D_IN), lambda i: (i, 0)),
            pl.BlockSpec((2, _ROWS, 16), lambda i: (0, i, 0)),
            pl.BlockSpec((_D_IN, _D_HID), lambda i: (0, 0)),
        ],
        out_specs=pl.BlockSpec((_ROWS, _D_HID), lambda i: (i, 0)),
        out_shape=jax.ShapeDtypeStruct((_NP, _D_HID), jnp.float32),
    )(xp, degp, W1)


def _mm2_body(p0_ref, p1_ref, dp_ref, b1_ref, w_ref, o_ref):
    ns, nd = _norms(dp_ref)
    agg = jnp.concatenate([p0_ref[0], p1_ref[0]], axis=1)
    h = jax.nn.relu(agg * nd + b1_ref[...])
    o_ref[...] = jnp.dot(h * ns, w_ref[...],
                         preferred_element_type=jnp.float32)


def _mm2(agg1, degp, b1, W2p):
    return pl.pallas_call(
        _mm2_body,
        grid=(_NP // _ROWS,),
        in_specs=[
            pl.BlockSpec((1, _ROWS, _D_HID // 2), lambda i: (0, i, 0)),
            pl.BlockSpec((1, _ROWS, _D_HID // 2), lambda i: (1, i, 0)),
            pl.BlockSpec((2, _ROWS, 16), lambda i: (0, i, 0)),
            pl.BlockSpec((1, _D_HID), lambda i: (0, 0)),
            pl.BlockSpec((_D_HID, _D_PAD), lambda i: (0, 0)),
        ],
        out_specs=pl.BlockSpec((_ROWS, _D_PAD), lambda i: (i, 0)),
        out_shape=jax.ShapeDtypeStruct((_NP, _D_PAD), jnp.float32),
    )(agg1, agg1, degp, b1, W2p)


def _fin_body(p0_ref, p1_ref, dp_ref, b2_ref, o_ref):
    _, nd = _norms(dp_ref)
    y = (p0_ref[0] + p1_ref[0]) * nd + b2_ref[...]
    o_ref[...] = y[:, :_D_OUT]


def _fin(agg2, degp, b2p):
    blk = _N // 10
    return pl.pallas_call(
        _fin_body,
        grid=(10,),
        in_specs=[
            pl.BlockSpec((1, blk, _D_PAD), lambda i: (0, i, 0)),
            pl.BlockSpec((1, blk, _D_PAD), lambda i: (1, i, 0)),
            pl.BlockSpec((2, blk, 16), lambda i: (0, i, 0)),
            pl.BlockSpec((1, _D_PAD), lambda i: (0, 0)),
        ],
        out_specs=pl.BlockSpec((blk, _D_OUT), lambda i: (i, 0)),
        out_shape=jax.ShapeDtypeStruct((_N, _D_OUT), jnp.float32),
    )(agg2, agg2, degp, b2p)


def _zero_tile_buf(zb, d):
    z = jnp.zeros((16,), jnp.float32)
    for r in range(16):
        for k in range(d // 16):
            zb[r, pl.ds(k * 16, 16)] = z


def _zero_acc_slice(zb, acc, base):
    def body(i, carry):
        pltpu.sync_copy(zb, acc.at[pl.ds(base + i * 16, 16)])
        return carry
    lax.fori_loop(0, _RPS // 16, body, 0)


def _spmm_body(table_h, sidx, didx, bufs, acc, gsems, ssems, nch):
    def gather(j, b):
        return pltpu.async_copy(table_h.at[sidx.at[j]], bufs[b], gsems[b])

    def gwait(j, b):
        pltpu.make_async_copy(table_h.at[sidx.at[j]], bufs[b],
                              gsems[b]).wait()

    def scatter(j, b):
        return pltpu.async_copy(bufs[b], acc.at[didx.at[j]], ssems[b],
                                add=True)

    def swait(j, b):
        pltpu.make_async_copy(bufs[b], acc.at[didx.at[j]], ssems[b]).wait()

    for b in range(_NBUF):
        gather(b, b)

    def group(g, carry):
        for b in range(_NBUF):
            j = g * _NBUF + b
            gwait(j, b)
            scatter(j, b)
        for b in range(_NBUF):
            j = g * _NBUF + b
            swait(j, b)
            nj = j + _NBUF

            @pl.when(nj < nch)
            def _():
                gather(nj, b)
        return carry

    lax.fori_loop(0, nch // _NBUF, group, 0)


def _spmm1_sc():
    nch = _EP // 16 // _C

    def wrapped(table_h, src_h, dst_h, out_h, sidx, didx, *rest):
        bufs, zb, acc = rest[:_NBUF], rest[_NBUF], rest[_NBUF + 1]
        gsems = rest[_NBUF + 2:2 * _NBUF + 2]
        ssems = rest[2 * _NBUF + 2:]
        c = lax.axis_index("c")
        s = lax.axis_index("s")
        half = nch // 2
        pltpu.sync_copy(src_h.at[2 * s], sidx.at[pl.ds(0, half)])
        pltpu.sync_copy(src_h.at[2 * s + 1], sidx.at[pl.ds(half, half)])
        pltpu.sync_copy(dst_h.at[2 * s], didx.at[pl.ds(0, half)])
        pltpu.sync_copy(dst_h.at[2 * s + 1], didx.at[pl.ds(half, half)])
        _zero_tile_buf(zb, 64)
        _zero_acc_slice(zb, acc, s * _RPS)

        def xrow(r, carry):
            for k in range(_C // 16):
                sl = pl.ds(k * 16, 16)
                sidx[r, sl] = sidx[r, sl] * 2 + c
            return carry

        lax.fori_loop(0, nch, xrow, 0)
        plsc.subcore_barrier()
        _spmm_body(table_h, sidx, didx, bufs, acc, gsems, ssems, nch)
        plsc.subcore_barrier()
        pltpu.sync_copy(acc.at[pl.ds(s * _RPS, _RPS)],
                        out_h.at[c, pl.ds(s * _RPS, _RPS)])

    return pl.kernel(
        wrapped,
        out_type=jax.ShapeDtypeStruct((2, _NP, 64), jnp.float32),
        mesh=_mesh(),
        compiler_params=pltpu.CompilerParams(use_tc_tiling_on_sc=False),
        scratch_types=[
            pltpu.VMEM((nch, _C), jnp.int32),
            pltpu.VMEM((nch, _C), jnp.int32),
            *[pltpu.VMEM((_C, 64), jnp.float32) for _ in range(_NBUF)],
            pltpu.VMEM((16, 64), jnp.float32),
            pltpu.VMEM_SHARED((_NP, 64), jnp.float32),
            *[pltpu.SemaphoreType.DMA for _ in range(2 * _NBUF)],
        ],
    )


def _spmm2_sc():
    nch = _EP // 32 // _C

    def wrapped(table_h, src_h, dst_h, out_h, sidx, didx, *rest):
        bufs, zb, acc = rest[:_NBUF], rest[_NBUF], rest[_NBUF + 1]
        gsems = rest[_NBUF + 2:2 * _NBUF + 2]
        ssems = rest[2 * _NBUF + 2:]
        c = lax.axis_index("c")
        s = lax.axis_index("s")
        w = c * 16 + s
        pltpu.sync_copy(src_h.at[w], sidx)
        pltpu.sync_copy(dst_h.at[w], didx)
        _zero_tile_buf(zb, _D_PAD)
        _zero_acc_slice(zb, acc, s * _RPS)
        plsc.subcore_barrier()
        _spmm_body(table_h, sidx, didx, bufs, acc, gsems, ssems, nch)
        plsc.subcore_barrier()
        pltpu.sync_copy(acc.at[pl.ds(s * _RPS, _RPS)],
                        out_h.at[c, pl.ds(s * _RPS, _RPS)])

    return pl.kernel(
        wrapped,
        out_type=jax.ShapeDtypeStruct((2, _NP, _D_PAD), jnp.float32),
        mesh=_mesh(),
        compiler_params=pltpu.CompilerParams(use_tc_tiling_on_sc=False),
        scratch_types=[
            pltpu.VMEM((nch, _C), jnp.int32),
            pltpu.VMEM((nch, _C), jnp.int32),
            *[pltpu.VMEM((_C, _D_PAD), jnp.float32) for _ in range(_NBUF)],
            pltpu.VMEM((16, _D_PAD), jnp.float32),
            pltpu.VMEM_SHARED((_NP, _D_PAD), jnp.float32),
            *[pltpu.SemaphoreType.DMA for _ in range(2 * _NBUF)],
        ],
    )


def _deg_sc():
    nch = _EP // 32 // _C

    def body(src_h, dst_h, out_h, sidx, didx, obs, obd, zb, acc, *sems):
        c = lax.axis_index("c")
        s = lax.axis_index("s")
        w = c * 16 + s

        _zero_tile_buf(zb, 16)
        lane = lax.iota(jnp.int32, 16)
        e0 = jnp.where(lane == 0, 1.0, 0.0)
        e1 = jnp.where(lane == 1, 1.0, 0.0)
        for r in range(_C):
            obs[r, pl.ds(0, 16)] = e0
            obd[r, pl.ds(0, 16)] = e1
        _zero_acc_slice(zb, acc, s * _RPS)

        pltpu.sync_copy(src_h.at[w], sidx)
        pltpu.sync_copy(dst_h.at[w], didx)
        plsc.subcore_barrier()

        def chunk(j, carry):
            @pl.when(j > 0)
            def _():
                pltpu.make_async_copy(obs, acc.at[sidx.at[j - 1]],
                                      sems[0]).wait()
                pltpu.make_async_copy(obd, acc.at[didx.at[j - 1]],
                                      sems[1]).wait()
            pltpu.async_copy(obs, acc.at[sidx.at[j]], sems[0], add=True)
            pltpu.async_copy(obd, acc.at[didx.at[j]], sems[1], add=True)
            return carry

        lax.fori_loop(0, nch, chunk, 0)
        pltpu.make_async_copy(obs, acc.at[sidx.at[nch - 1]], sems[0]).wait()
        pltpu.make_async_copy(obd, acc.at[didx.at[nch - 1]], sems[1]).wait()

        plsc.subcore_barrier()
        sl = pl.ds(s * _RPS, _RPS)
        pltpu.sync_copy(acc.at[sl], out_h.at[c, sl])

    return pl.kernel(
        body,
        out_type=jax.ShapeDtypeStruct((2, _NP, 16), jnp.float32),
        mesh=_mesh(),
        compiler_params=pltpu.CompilerParams(use_tc_tiling_on_sc=False),
        scratch_types=[
            pltpu.VMEM((nch, _C), jnp.int32),
            pltpu.VMEM((nch, _C), jnp.int32),
            pltpu.VMEM((_C, 16), jnp.float32),
            pltpu.VMEM((_C, 16), jnp.float32),
            pltpu.VMEM((16, 16), jnp.float32),
            pltpu.VMEM_SHARED((_NP, 16), jnp.float32),
            pltpu.SemaphoreType.DMA,
            pltpu.SemaphoreType.DMA,
        ],
    )


def kernel(x, edge_index, W1, b1, W2, b2):
    src = edge_index[0]
    dst = edge_index[1]
    pad_rows = _N + jnp.arange(_EP - _E, dtype=jnp.int32) % (_NP - _N)
    srcp = jnp.concatenate([src, pad_rows])
    dstp = jnp.concatenate([dst, pad_rows])
    src32 = srcp.reshape(32, _EP // 32 // _C, _C)
    dst32 = dstp.reshape(32, _EP // 32 // _C, _C)

    degp = _deg_sc()(src32, dst32)

    t1 = _mm1(x, degp, W1)
    agg1 = _spmm1_sc()(t1.reshape(2 * _NP, 64), src32, dst32)

    W2p = jnp.pad(W2, ((0, 0), (0, _D_PAD - _D_OUT)))
    t2 = _mm2(agg1, degp, b1[None, :], W2p)
    agg2 = _spmm2_sc()(t2, src32, dst32)

    b2p = jnp.pad(b2, (0, _D_PAD - _D_OUT))
    out = _fin(agg2, degp, b2p[None, :])
    return out[:_N, :_D_OUT]


# speedup vs baseline: 13.9295x; 1.0117x over previous
import jax
import jax.numpy as jnp
from jax import lax
from jax.experimental import pallas as pl
from jax.experimental.pallas import tpu as pltpu
from jax.experimental.pallas import tpu_sc as plsc

_N = 10000
_E = 320000
_D_IN = 128
_D_HID = 128
_D_OUT = 40
_D_PAD = 48

_NP = 10240
_C = 128
_EP = 327680
_NBUF = 5
_RPS = _NP // 16

_ROWS = 1024


def _mesh():
    return plsc.VectorSubcoreMesh(core_axis_name="c", subcore_axis_name="s",
                                  num_cores=2, num_subcores=16)


def _norms(dp_ref):
    d = dp_ref[0] + dp_ref[1]
    ns = d[:, 0:1]
    nd = d[:, 1:2]
    ns = jnp.where(ns > 0, lax.rsqrt(ns), 0.0)
    nd = jnp.where(nd > 0, lax.rsqrt(nd), 0.0)
    return ns, nd


def _mm1_body(x_ref, dp_ref, w_ref, o_ref):
    ns, _ = _norms(dp_ref)
    o_ref[...] = jnp.dot(x_ref[...] * ns, w_ref[...],
                         preferred_element_type=jnp.float32)


def _mm1(xp, degp, W1):
    return pl.pallas_call(
        _mm1_body,
        grid=(_NP // _ROWS,),
        in_specs=[
            pl.BlockSpec((_ROWS, _D_IN), lambda i: (i, 0)),
            pl.BlockSpec((2, _ROWS, 16), lambda i: (0, i, 0)),
            pl.BlockSpec((_D_IN, _D_HID), lambda i: (0, 0)),
        ],
        out_specs=pl.BlockSpec((_ROWS, _D_HID), lambda i: (i, 0)),
        out_shape=jax.ShapeDtypeStruct((_NP, _D_HID), jnp.float32),
    )(xp, degp, W1)


def _mm2_body(p0_ref, p1_ref, dp_ref, b1_ref, w_ref, o_ref):
    ns, nd = _norms(dp_ref)
    agg = jnp.concatenate([p0_ref[0], p1_ref[0]], axis=1)
    h = jax.nn.relu(agg * nd + b1_ref[...])
    o_ref[...] = jnp.dot(h * ns, w_ref[...],
                         preferred_element_type=jnp.float32)


def _mm2(agg1, degp, b1, W2p):
    return pl.pallas_call(
        _mm2_body,
        grid=(_NP // _ROWS,),
        in_specs=[
            pl.BlockSpec((1, _ROWS, _D_HID // 2), lambda i: (0, i, 0)),
            pl.BlockSpec((1, _ROWS, _D_HID // 2), lambda i: (1, i, 0)),
            pl.BlockSpec((2, _ROWS, 16), lambda i: (0, i, 0)),
            pl.BlockSpec((1, _D_HID), lambda i: (0, 0)),
            pl.BlockSpec((_D_HID, _D_PAD), lambda i: (0, 0)),
        ],
        out_specs=pl.BlockSpec((_ROWS, _D_PAD), lambda i: (i, 0)),
        out_shape=jax.ShapeDtypeStruct((_NP, _D_PAD), jnp.float32),
    )(agg1, agg1, degp, b1, W2p)


def _fin_body(p0_ref, p1_ref, dp_ref, b2_ref, o_ref):
    _, nd = _norms(dp_ref)
    y = (p0_ref[0] + p1_ref[0]) * nd + b2_ref[...]
    o_ref[...] = y[:, :_D_OUT]


def _fin(agg2, degp, b2p):
    blk = _N // 10
    return pl.pallas_call(
        _fin_body,
        grid=(10,),
        in_specs=[
            pl.BlockSpec((1, blk, _D_PAD), lambda i: (0, i, 0)),
            pl.BlockSpec((1, blk, _D_PAD), lambda i: (1, i, 0)),
            pl.BlockSpec((2, blk, 16), lambda i: (0, i, 0)),
            pl.BlockSpec((1, _D_PAD), lambda i: (0, 0)),
        ],
        out_specs=pl.BlockSpec((blk, _D_OUT), lambda i: (i, 0)),
        out_shape=jax.ShapeDtypeStruct((_N, _D_OUT), jnp.float32),
    )(agg2, agg2, degp, b2p)


def _zero_tile_buf(zb, d):
    z = jnp.zeros((16,), jnp.float32)
    for r in range(16):
        for k in range(d // 16):
            zb[r, pl.ds(k * 16, 16)] = z


def _zero_acc_slice(zb, acc, base):
    def body(i, carry):
        pltpu.sync_copy(zb, acc.at[pl.ds(base + i * 16, 16)])
        return carry
    lax.fori_loop(0, _RPS // 16, body, 0)


def _spmm_body(table_h, sidx, didx, bufs, acc, gsems, ssems, nch, prep=None):
    def gather(j, b):
        return pltpu.async_copy(table_h.at[sidx.at[j]], bufs[b], gsems[b])

    def gwait(j, b):
        pltpu.make_async_copy(table_h.at[sidx.at[j]], bufs[b],
                              gsems[b]).wait()

    def scatter(j, b):
        return pltpu.async_copy(bufs[b], acc.at[didx.at[j]], ssems[b],
                                add=True)

    def swait(j, b):
        pltpu.make_async_copy(bufs[b], acc.at[didx.at[j]], ssems[b]).wait()

    for b in range(_NBUF):
        if prep is not None:
            prep(b)
        gather(b, b)

    def group(g, carry):
        for b in range(_NBUF):
            j = g * _NBUF + b
            gwait(j, b)
            scatter(j, b)
        for b in range(_NBUF):
            j = g * _NBUF + b
            swait(j, b)
            nj = j + _NBUF

            @pl.when(nj < nch)
            def _():
                if prep is not None:
                    prep(nj)
                gather(nj, b)
        return carry

    lax.fori_loop(0, nch // _NBUF, group, 0)


def _spmm1_sc():
    nch = _EP // 16 // _C

    def wrapped(table_h, src_h, dst_h, out_h, sidx, didx, *rest):
        bufs, zb, acc = rest[:_NBUF], rest[_NBUF], rest[_NBUF + 1]
        gsems = rest[_NBUF + 2:2 * _NBUF + 2]
        ssems = rest[2 * _NBUF + 2:]
        c = lax.axis_index("c")
        s = lax.axis_index("s")
        half = nch // 2
        pltpu.sync_copy(src_h.at[2 * s], sidx.at[pl.ds(0, half)])
        pltpu.sync_copy(src_h.at[2 * s + 1], sidx.at[pl.ds(half, half)])
        pltpu.sync_copy(dst_h.at[2 * s], didx.at[pl.ds(0, half)])
        pltpu.sync_copy(dst_h.at[2 * s + 1], didx.at[pl.ds(half, half)])
        _zero_tile_buf(zb, 64)
        _zero_acc_slice(zb, acc, s * _RPS)

        def xrow(r):
            for k in range(_C // 16):
                sl = pl.ds(k * 16, 16)
                sidx[r, sl] = sidx[r, sl] * 2 + c

        plsc.subcore_barrier()
        _spmm_body(table_h, sidx, didx, bufs, acc, gsems, ssems, nch,
                   prep=xrow)
        plsc.subcore_barrier()
        pltpu.sync_copy(acc.at[pl.ds(s * _RPS, _RPS)],
                        out_h.at[c, pl.ds(s * _RPS, _RPS)])

    return pl.kernel(
        wrapped,
        out_type=jax.ShapeDtypeStruct((2, _NP, 64), jnp.float32),
        mesh=_mesh(),
        compiler_params=pltpu.CompilerParams(use_tc_tiling_on_sc=False),
        scratch_types=[
            pltpu.VMEM((nch, _C), jnp.int32),
            pltpu.VMEM((nch, _C), jnp.int32),
            *[pltpu.VMEM((_C, 64), jnp.float32) for _ in range(_NBUF)],
            pltpu.VMEM((16, 64), jnp.float32),
            pltpu.VMEM_SHARED((_NP, 64), jnp.float32),
            *[pltpu.SemaphoreType.DMA for _ in range(2 * _NBUF)],
        ],
    )


def _spmm2_sc():
    nch = _EP // 32 // _C

    def wrapped(table_h, src_h, dst_h, out_h, sidx, didx, *rest):
        bufs, zb, acc = rest[:_NBUF], rest[_NBUF], rest[_NBUF + 1]
        gsems = rest[_NBUF + 2:2 * _NBUF + 2]
        ssems = rest[2 * _NBUF + 2:]
        c = lax.axis_index("c")
        s = lax.axis_index("s")
        w = c * 16 + s
        pltpu.sync_copy(src_h.at[w], sidx)
        pltpu.sync_copy(dst_h.at[w], didx)
        _zero_tile_buf(zb, _D_PAD)
        _zero_acc_slice(zb, acc, s * _RPS)
        plsc.subcore_barrier()
        _spmm_body(table_h, sidx, didx, bufs, acc, gsems, ssems, nch)
        plsc.subcore_barrier()
        pltpu.sync_copy(acc.at[pl.ds(s * _RPS, _RPS)],
                        out_h.at[c, pl.ds(s * _RPS, _RPS)])

    return pl.kernel(
        wrapped,
        out_type=jax.ShapeDtypeStruct((2, _NP, _D_PAD), jnp.float32),
        mesh=_mesh(),
        compiler_params=pltpu.CompilerParams(use_tc_tiling_on_sc=False),
        scratch_types=[
            pltpu.VMEM((nch, _C), jnp.int32),
            pltpu.VMEM((nch, _C), jnp.int32),
            *[pltpu.VMEM((_C, _D_PAD), jnp.float32) for _ in range(_NBUF)],
            pltpu.VMEM((16, _D_PAD), jnp.float32),
            pltpu.VMEM_SHARED((_NP, _D_PAD), jnp.float32),
            *[pltpu.SemaphoreType.DMA for _ in range(2 * _NBUF)],
        ],
    )


def _deg_sc():
    nch = _EP // 32 // _C

    def body(src_h, dst_h, out_h, sidx, didx, obs, obd, zb, acc, *sems):
        c = lax.axis_index("c")
        s = lax.axis_index("s")
        w = c * 16 + s

        _zero_tile_buf(zb, 16)
        lane = lax.iota(jnp.int32, 16)
        e0 = jnp.where(lane == 0, 1.0, 0.0)
        e1 = jnp.where(lane == 1, 1.0, 0.0)
        for r in range(_C):
            obs[r, pl.ds(0, 16)] = e0
            obd[r, pl.ds(0, 16)] = e1
        _zero_acc_slice(zb, acc, s * _RPS)

        pltpu.sync_copy(src_h.at[w], sidx)
        pltpu.sync_copy(dst_h.at[w], didx)
        plsc.subcore_barrier()

        def chunk(j, carry):
            @pl.when(j > 1)
            def _():
                pltpu.make_async_copy(obs, acc.at[sidx.at[j - 2]],
                                      sems[0]).wait()
                pltpu.make_async_copy(obd, acc.at[didx.at[j - 2]],
                                      sems[1]).wait()
            pltpu.async_copy(obs, acc.at[sidx.at[j]], sems[0], add=True)
            pltpu.async_copy(obd, acc.at[didx.at[j]], sems[1], add=True)
            return carry

        lax.fori_loop(0, nch, chunk, 0)
        for jt in (nch - 2, nch - 1):
            pltpu.make_async_copy(obs, acc.at[sidx.at[jt]], sems[0]).wait()
            pltpu.make_async_copy(obd, acc.at[didx.at[jt]], sems[1]).wait()

        plsc.subcore_barrier()
        sl = pl.ds(s * _RPS, _RPS)
        pltpu.sync_copy(acc.at[sl], out_h.at[c, sl])

    return pl.kernel(
        body,
        out_type=jax.ShapeDtypeStruct((2, _NP, 16), jnp.float32),
        mesh=_mesh(),
        compiler_params=pltpu.CompilerParams(use_tc_tiling_on_sc=False),
        scratch_types=[
            pltpu.VMEM((nch, _C), jnp.int32),
            pltpu.VMEM((nch, _C), jnp.int32),
            pltpu.VMEM((_C, 16), jnp.float32),
            pltpu.VMEM((_C, 16), jnp.float32),
            pltpu.VMEM((16, 16), jnp.float32),
            pltpu.VMEM_SHARED((_NP, 16), jnp.float32),
            pltpu.SemaphoreType.DMA,
            pltpu.SemaphoreType.DMA,
        ],
    )


def kernel(x, edge_index, W1, b1, W2, b2):
    src = edge_index[0]
    dst = edge_index[1]
    pad_rows = _N + jnp.arange(_EP - _E, dtype=jnp.int32) % (_NP - _N)
    srcp = jnp.concatenate([src, pad_rows])
    dstp = jnp.concatenate([dst, pad_rows])
    src32 = srcp.reshape(32, _EP // 32 // _C, _C)
    dst32 = dstp.reshape(32, _EP // 32 // _C, _C)

    degp = _deg_sc()(src32, dst32)

    t1 = _mm1(x, degp, W1)
    agg1 = _spmm1_sc()(t1.reshape(2 * _NP, 64), src32, dst32)

    W2p = jnp.pad(W2, ((0, 0), (0, _D_PAD - _D_OUT)))
    t2 = _mm2(agg1, degp, b1[None, :], W2p)
    agg2 = _spmm2_sc()(t2, src32, dst32)

    b2p = jnp.pad(b2, (0, _D_PAD - _D_OUT))
    out = _fin(agg2, degp, b2p[None, :])
    return out[:_N, :_D_OUT]
